# baseline (speedup 1.0000x reference)
import jax
import jax.numpy as jnp
from jax import lax
from jax.experimental import pallas as pl
from jax.experimental.pallas import tpu as pltpu
from jax.experimental.pallas import tpu_sc as plsc

VOCAB = 1_000_000
D = 64
T = 200
B = 4096
FLAT = B * T
NC = 2
NS = 16
NW = NC * NS
PER_W = FLAT // NW
CHUNK = 128
NCHUNK = PER_W // CHUNK
NBUF = 4
NGRP = NCHUNK // NBUF
GROUP = NBUF * CHUNK
LANES = 16
KD = D // LANES


def _body(idx_hbm, tok_hbm, pos_hbm, out_hbm,
          ix0, ix1, q0, q1, rows_v, ob0, ob1, pat_v,
          sgi0, sgi1, sg0, sg1, sg2, sg3, so0, so1):
    cid = lax.axis_index("c")
    sid = lax.axis_index("s")
    wid = sid * NC + cid
    w0 = wid * PER_W
    ixg = [ix0, ix1]
    qg = [q0, q1]
    sgi = [sgi0, sgi1]
    sg = [sg0, sg1, sg2, sg3]
    so = [so0, so1]
    outb = [ob0, ob1]

    pltpu.sync_copy(pos_hbm, pat_v)

    def idx_start(g, p):
        pltpu.async_copy(idx_hbm.at[pl.ds(w0 + g * GROUP, GROUP)], ixg[p], sgi[p])

    def idx_wait(g, p):
        pltpu.make_async_copy(
            idx_hbm.at[pl.ds(w0 + g * GROUP, GROUP)], ixg[p], sgi[p]).wait()
        for v in range(GROUP // LANES):
            sl = pl.ds(v * LANES, LANES)
            qg[p][sl] = lax.shift_right_logical(ixg[p][sl], 1)

    def gather_start(b, p):
        pltpu.async_copy(tok_hbm.at[qg[p].at[pl.ds(b * CHUNK, CHUNK)]],
                         rows_v.at[b], sg[b])

    def gather_wait(b):
        pltpu.make_async_copy(tok_hbm.at[pl.ds(0, CHUNK)], rows_v.at[b], sg[b]).wait()

    def out_start(ci, ob):
        pltpu.async_copy(outb[ob], out_hbm.at[pl.ds(w0 + ci * CHUNK, CHUNK)], so[ob])

    def out_wait(ci, ob):
        pltpu.make_async_copy(
            outb[ob], out_hbm.at[pl.ds(w0 + ci * CHUNK, CHUNK)], so[ob]).wait()

    def extract_add(ci, b, ob, pv):
        base = lax.rem(w0 + ci * CHUNK, T)

        def m_body(m, _):
            r0 = m * LANES
            iv = ixg[pv][pl.ds(b * CHUNK + r0, LANES)]
            offv = jnp.bitwise_and(iv, 1) * D
            for l in range(LANES):
                off = offv[l]
                r = r0 + l
                tt = lax.rem(base + r, T)
                for k in range(KD):
                    sl = pl.ds(k * LANES, LANES)
                    outb[ob][r, sl] = (rows_v[b, r, pl.ds(off + k * LANES, LANES)]
                                       + pat_v[tt, sl])
            return ()

        lax.fori_loop(0, CHUNK // LANES, m_body, ())

    idx_start(0, 0)
    idx_wait(0, 0)
    for b in range(NBUF):
        gather_start(b, 0)

    def phase(g, pv):
        pn = (pv + 1) % 2

        @pl.when(g < NGRP - 1)
        def _():
            idx_start(g + 1, pn)

        for b in range(NBUF):
            ci = NBUF * g + b
            ob = b % 2
            gather_wait(b)
            if b < 2:
                @pl.when(g > 0)
                def _():
                    out_wait(ci - 2, ob)
            else:
                out_wait(ci - 2, ob)
            extract_add(ci, b, ob, pv)
            out_start(ci, ob)

            @pl.when(g < NGRP - 1)
            def _():
                if b == 0:
                    idx_wait(g + 1, pn)
                gather_start(b, pn)

    def g_body(go, _):
        phase(2 * go, 0)
        phase(2 * go + 1, 1)
        return ()

    lax.fori_loop(0, NGRP // 2, g_body, ())
    out_wait(NCHUNK - 2, 0)
    out_wait(NCHUNK - 1, 1)


@jax.jit
def _embed(idx1d, tokp, posp):
    mesh = plsc.VectorSubcoreMesh(core_axis_name="c", subcore_axis_name="s")
    f = pl.kernel(
        _body,
        mesh=mesh,
        out_type=jax.ShapeDtypeStruct((FLAT, D), jnp.float32),
        scratch_types=[
            pltpu.VMEM((GROUP,), jnp.int32),
            pltpu.VMEM((GROUP,), jnp.int32),
            pltpu.VMEM((GROUP,), jnp.int32),
            pltpu.VMEM((GROUP,), jnp.int32),
            pltpu.VMEM((NBUF, CHUNK, 2 * D), jnp.float32),
            pltpu.VMEM((CHUNK, D), jnp.float32),
            pltpu.VMEM((CHUNK, D), jnp.float32),
            pltpu.VMEM((T, 2 * D), jnp.float32),
        ] + [pltpu.SemaphoreType.DMA] * 8,
        compiler_params=pltpu.CompilerParams(use_tc_tiling_on_sc=True),
    )
    return f(idx1d, tokp, posp)


def kernel(idx, token_embedding_table, position_embedding_table):
    idx1d = idx.astype(jnp.int32).reshape(FLAT)
    tok2 = token_embedding_table.reshape(VOCAB // 2, 2 * D)
    posp = jnp.pad(position_embedding_table, ((0, 0), (0, D)))
    out = _embed(idx1d, tok2, posp)
    return out.reshape(B, T, D)


# speedup vs baseline: 1.3598x; 1.3598x over previous
import jax
import jax.numpy as jnp
from jax import lax
from jax.experimental import pallas as pl
from jax.experimental.pallas import tpu as pltpu
from jax.experimental.pallas import tpu_sc as plsc

VOCAB = 1_000_000
D = 64
T = 200
B = 4096
FLAT = B * T
NC = 2
NS = 16
NW = NC * NS
PER_W = FLAT // NW
CHUNK = 128
NCHUNK = PER_W // CHUNK
NBUF = 4
NGRP = NCHUNK // NBUF
GROUP = NBUF * CHUNK
LANES = 16
KD = D // LANES


def _body(idx_hbm, tok_hbm, pos_hbm, out_hbm,
          ix0, ix1, rows_v, ob0, ob1, pat_v,
          sgi0, sgi1, sg0, sg1, sg2, sg3, so0, so1):
    cid = lax.axis_index("c")
    sid = lax.axis_index("s")
    wid = sid * NC + cid
    w0 = wid * PER_W
    ixg = [ix0, ix1]
    sgi = [sgi0, sgi1]
    sg = [sg0, sg1, sg2, sg3]
    so = [so0, so1]
    outb = [ob0, ob1]

    pltpu.sync_copy(pos_hbm, pat_v)

    def idx_start(g, p):
        pltpu.async_copy(idx_hbm.at[pl.ds(w0 + g * GROUP, GROUP)], ixg[p], sgi[p])

    def idx_wait(g, p):
        pltpu.make_async_copy(
            idx_hbm.at[pl.ds(w0 + g * GROUP, GROUP)], ixg[p], sgi[p]).wait()

    def gather_start(b, p):
        pltpu.async_copy(tok_hbm.at[ixg[p].at[pl.ds(b * CHUNK, CHUNK)]],
                         rows_v.at[b], sg[b])

    def gather_wait(b):
        pltpu.make_async_copy(tok_hbm.at[pl.ds(0, CHUNK)], rows_v.at[b], sg[b]).wait()

    def out_start(ci, ob):
        pltpu.async_copy(outb[ob], out_hbm.at[pl.ds(w0 + ci * CHUNK, CHUNK)], so[ob])

    def out_wait(ci, ob):
        pltpu.make_async_copy(
            outb[ob], out_hbm.at[pl.ds(w0 + ci * CHUNK, CHUNK)], so[ob]).wait()

    def extract_add(ci, b, ob):
        base = lax.rem(w0 + ci * CHUNK, T)

        @plsc.parallel_loop(0, CHUNK, step=1, unroll=8)
        def _(r):
            tt = lax.rem(base + r, T)
            for k in range(KD):
                sl = pl.ds(k * LANES, LANES)
                outb[ob][r, sl] = rows_v[b, r, sl] + pat_v[tt, sl]

    idx_start(0, 0)
    idx_wait(0, 0)
    for b in range(NBUF):
        gather_start(b, 0)

    def phase(g, pv):
        pn = (pv + 1) % 2

        @pl.when(g < NGRP - 1)
        def _():
            idx_start(g + 1, pn)

        for b in range(NBUF):
            ci = NBUF * g + b
            ob = b % 2
            gather_wait(b)
            if b < 2:
                @pl.when(g > 0)
                def _():
                    out_wait(ci - 2, ob)
            else:
                out_wait(ci - 2, ob)
            extract_add(ci, b, ob)
            out_start(ci, ob)

            @pl.when(g < NGRP - 1)
            def _():
                if b == 0:
                    idx_wait(g + 1, pn)
                gather_start(b, pn)

    def g_body(go, _):
        phase(2 * go, 0)
        phase(2 * go + 1, 1)
        return ()

    lax.fori_loop(0, NGRP // 2, g_body, ())
    out_wait(NCHUNK - 2, 0)
    out_wait(NCHUNK - 1, 1)


@jax.jit
def _embed(idx1d, tokp, posp):
    mesh = plsc.VectorSubcoreMesh(core_axis_name="c", subcore_axis_name="s")
    f = pl.kernel(
        _body,
        mesh=mesh,
        out_type=jax.ShapeDtypeStruct((FLAT, D), jnp.float32),
        scratch_types=[
            pltpu.VMEM((GROUP,), jnp.int32),
            pltpu.VMEM((GROUP,), jnp.int32),
            pltpu.VMEM((NBUF, CHUNK, 2 * D), jnp.float32),
            pltpu.VMEM((CHUNK, D), jnp.float32),
            pltpu.VMEM((CHUNK, D), jnp.float32),
            pltpu.VMEM((T, 2 * D), jnp.float32),
        ] + [pltpu.SemaphoreType.DMA] * 8,
        compiler_params=pltpu.CompilerParams(use_tc_tiling_on_sc=True),
    )
    return f(idx1d, tokp, posp)


def kernel(idx, token_embedding_table, position_embedding_table):
    idx1d = idx.astype(jnp.int32).reshape(FLAT)
    tokp = jnp.pad(token_embedding_table, ((0, 0), (0, D)))
    posp = jnp.pad(position_embedding_table, ((0, 0), (0, D)))
    out = _embed(idx1d, tokp, posp)
    return out.reshape(B, T, D)
